# Initial kernel scaffold; baseline (speedup 1.0000x reference)
import jax
import jax.numpy as jnp
from jax.experimental import pallas as pl


def kernel(node_type, edge_index, edge_type, node_emb, W_rel, W_root, b):
    raise NotImplementedError("write your pallas kernel here")


# speedup vs baseline: 2.6911x; 2.6911x over previous
import functools

import jax
import jax.numpy as jnp
from jax import lax
from jax.experimental import pallas as pl
from jax.experimental.pallas import tpu as pltpu
from jax.experimental.pallas import tpu_sc as plsc

N = 10000
E = 320000
D = 128
R = 16
L = 3
NEG_SLOPE = 0.01

NC = 2
NS = 16
NW = NC * NS

GPW = 320
GCH = 64


def _emb_gather_body(nt_hbm, emb_hbm, out_hbm, idx_v, rows_v, sem):
    wid = lax.axis_index("s") * NC + lax.axis_index("c")
    base = pl.multiple_of(jnp.minimum(wid * GPW, N - GPW), 8)
    pltpu.sync_copy(nt_hbm.at[pl.ds(base, GPW)], idx_v)
    for c in range(GPW // GCH):
        pltpu.async_copy(
            emb_hbm.at[idx_v.at[pl.ds(c * GCH, GCH)]],
            rows_v.at[pl.ds(c * GCH, GCH)],
            sem,
        ).wait()
    pltpu.sync_copy(rows_v, out_hbm.at[pl.ds(base, GPW)])


_emb_gather = functools.partial(
    pl.kernel,
    out_type=jax.ShapeDtypeStruct((N, D), jnp.float32),
    mesh=plsc.VectorSubcoreMesh(core_axis_name="c", subcore_axis_name="s"),
    scratch_types=[
        pltpu.VMEM((GPW,), jnp.int32),
        pltpu.VMEM((GPW, D), jnp.float32),
        pltpu.SemaphoreType.DMA,
    ],
)(_emb_gather_body)


BM = 1000


def _leaky(s):
    return jnp.where(s >= 0, s, NEG_SLOPE * s)


def _mm0_body(x_ref, wr_ref, wo_ref, b_ref, xw_ref, rt_ref):
    x = x_ref[...]
    xw_ref[...] = jnp.dot(x, wr_ref[...], preferred_element_type=jnp.float32)
    rt_ref[...] = (
        jnp.dot(x, wo_ref[...], preferred_element_type=jnp.float32) + b_ref[...]
    )


def _mm1_body(a0_ref, a1_ref, rtin_ref, wr_ref, wo_ref, b_ref, xw_ref, rt_ref):
    x = _leaky(a0_ref[...] + a1_ref[...] + rtin_ref[...])
    xw_ref[...] = jnp.dot(x, wr_ref[...], preferred_element_type=jnp.float32)
    rt_ref[...] = (
        jnp.dot(x, wo_ref[...], preferred_element_type=jnp.float32) + b_ref[...]
    )


_row_spec = pl.BlockSpec((BM, D), lambda i: (i, 0))
_w_specs = [
    pl.BlockSpec((D, R * D), lambda i: (0, 0)),
    pl.BlockSpec((D, D), lambda i: (0, 0)),
    pl.BlockSpec((1, D), lambda i: (0, 0)),
]
_mm_out_shapes = [
    jax.ShapeDtypeStruct((N, R * D), jnp.float32),
    jax.ShapeDtypeStruct((N, D), jnp.float32),
]
_mm_out_specs = [
    pl.BlockSpec((BM, R * D), lambda i: (i, 0)),
    pl.BlockSpec((BM, D), lambda i: (i, 0)),
]

_mm0 = pl.pallas_call(
    _mm0_body,
    grid=(N // BM,),
    in_specs=[_row_spec] + _w_specs,
    out_specs=_mm_out_specs,
    out_shape=_mm_out_shapes,
)

_mm1 = pl.pallas_call(
    _mm1_body,
    grid=(N // BM,),
    in_specs=[_row_spec, _row_spec, _row_spec] + _w_specs,
    out_specs=_mm_out_specs,
    out_shape=_mm_out_shapes,
)


def _fin_body(a0_ref, a1_ref, rt_ref, o_ref):
    o_ref[...] = _leaky(a0_ref[...] + a1_ref[...] + rt_ref[...])


_fin = pl.pallas_call(
    _fin_body,
    grid=(N // BM,),
    in_specs=[_row_spec, _row_spec, _row_spec],
    out_specs=pl.BlockSpec((BM, D), lambda i: (i, 0)),
    out_shape=jax.ShapeDtypeStruct((N, D), jnp.float32),
)


CK = 80
EPW = E // NW
CPT = EPW // CK
RPT = 632


def _edge_body(table_hbm, src_hbm, et_hbm, dst_hbm, zeros_hbm, out_hbm,
               idxv, etv, dstv, rows_v, acc_sh, sem):
    cid = lax.axis_index("c")
    sid = lax.axis_index("s")
    wid = sid * NC + cid
    ebase = pl.multiple_of(wid * EPW, 8)
    pltpu.sync_copy(src_hbm.at[pl.ds(ebase, EPW)], idxv)
    pltpu.sync_copy(et_hbm.at[pl.ds(ebase, EPW)], etv)
    pltpu.sync_copy(dst_hbm.at[wid], dstv)
    rbase = pl.multiple_of(jnp.minimum(sid * RPT, N - RPT), 8)
    pltpu.sync_copy(
        zeros_hbm.at[pl.ds(rbase, RPT)], acc_sh.at[pl.ds(rbase, RPT)]
    )

    def _vec(i, carry):
        sl = pl.ds(pl.multiple_of(i * 16, 8), 16)
        idxv[sl] = idxv[sl] * R + etv[sl]
        return carry

    lax.fori_loop(0, EPW // 16, _vec, 0)
    plsc.subcore_barrier()

    def _chunk(c, carry):
        gsl = pl.ds(pl.multiple_of(c * CK, 8), CK)
        pltpu.async_copy(table_hbm.at[idxv.at[gsl]], rows_v, sem).wait()
        pltpu.sync_copy(rows_v, acc_sh.at[dstv.at[c]], add=True)
        return carry

    lax.fori_loop(0, CPT, _chunk, 0)
    plsc.subcore_barrier()
    pltpu.sync_copy(
        acc_sh.at[pl.ds(rbase, RPT)], out_hbm.at[cid, pl.ds(rbase, RPT)]
    )


_edge_agg = functools.partial(
    pl.kernel,
    out_type=jax.ShapeDtypeStruct((NC, N, D), jnp.float32),
    mesh=plsc.VectorSubcoreMesh(core_axis_name="c", subcore_axis_name="s"),
    scratch_types=[
        pltpu.VMEM((EPW,), jnp.int32),
        pltpu.VMEM((EPW,), jnp.int32),
        pltpu.VMEM((CPT, CK), jnp.int32),
        pltpu.VMEM((CK, D), jnp.float32),
        pltpu.VMEM_SHARED((N, D), jnp.float32),
        pltpu.SemaphoreType.DMA,
    ],
)(_edge_body)


def kernel(node_type, edge_index, edge_type, node_emb, W_rel, W_root, b):
    node_type = node_type.astype(jnp.int32)
    src = edge_index[0].astype(jnp.int32)
    dst = edge_index[1].astype(jnp.int32).reshape(NW, CPT, CK)
    et = edge_type.astype(jnp.int32)
    zeros = jnp.zeros((N, D), jnp.float32)

    x0 = _emb_gather(node_type, node_emb)

    rt = None
    aggs = None
    for l in range(L):
        wr = W_rel[l].transpose(1, 0, 2).reshape(D, R * D)
        wo = W_root[l]
        bl = b[l].reshape(1, D)
        if l == 0:
            xw, rt = _mm0(x0, wr, wo, bl)
        else:
            xw, rt = _mm1(aggs[0], aggs[1], rt, wr, wo, bl)
        aggs = _edge_agg(xw.reshape(N * R, D), src, et, dst, zeros)

    return _fin(aggs[0], aggs[1], rt)


# speedup vs baseline: 3.2878x; 1.2217x over previous
import functools

import jax
import jax.numpy as jnp
from jax import lax
from jax.experimental import pallas as pl
from jax.experimental.pallas import tpu as pltpu
from jax.experimental.pallas import tpu_sc as plsc

N = 10000
E = 320000
D = 128
R = 16
TPAD = 128
L = 3
NEG_SLOPE = 0.01

NC = 2
NS = 16
NW = NC * NS

BM = 1000


def _leaky(s):
    return jnp.where(s >= 0, s, NEG_SLOPE * s)


def _mm0_body(nt_ref, emb_ref, wr_ref, wo_ref, b_ref, xw_ref, rt_ref):
    nt = nt_ref[0, 0, :]
    tids = lax.broadcasted_iota(jnp.int32, (BM, TPAD), 1)
    onehot = (tids == nt[:, None]).astype(jnp.float32)
    x = jnp.dot(onehot, emb_ref[...], preferred_element_type=jnp.float32)
    xw_ref[...] = jnp.dot(x, wr_ref[...], preferred_element_type=jnp.float32)
    rt_ref[...] = (
        jnp.dot(x, wo_ref[...], preferred_element_type=jnp.float32) + b_ref[...]
    )


def _mm1_body(a0_ref, a1_ref, rtin_ref, wr_ref, wo_ref, b_ref, xw_ref, rt_ref):
    x = _leaky(a0_ref[...] + a1_ref[...] + rtin_ref[...])
    xw_ref[...] = jnp.dot(x, wr_ref[...], preferred_element_type=jnp.float32)
    rt_ref[...] = (
        jnp.dot(x, wo_ref[...], preferred_element_type=jnp.float32) + b_ref[...]
    )


_row_spec = pl.BlockSpec((BM, D), lambda i: (i, 0))
_w_specs = [
    pl.BlockSpec((D, R * D), lambda i: (0, 0)),
    pl.BlockSpec((D, D), lambda i: (0, 0)),
    pl.BlockSpec((1, D), lambda i: (0, 0)),
]
_mm_out_shapes = [
    jax.ShapeDtypeStruct((N, R * D), jnp.float32),
    jax.ShapeDtypeStruct((N, D), jnp.float32),
]
_mm_out_specs = [
    pl.BlockSpec((BM, R * D), lambda i: (i, 0)),
    pl.BlockSpec((BM, D), lambda i: (i, 0)),
]

_mm0 = pl.pallas_call(
    _mm0_body,
    grid=(N // BM,),
    in_specs=[
        pl.BlockSpec((1, 1, BM), lambda i: (i, 0, 0)),
        pl.BlockSpec((TPAD, D), lambda i: (0, 0)),
    ] + _w_specs,
    out_specs=_mm_out_specs,
    out_shape=_mm_out_shapes,
)

_mm1 = pl.pallas_call(
    _mm1_body,
    grid=(N // BM,),
    in_specs=[_row_spec, _row_spec, _row_spec] + _w_specs,
    out_specs=_mm_out_specs,
    out_shape=_mm_out_shapes,
)


def _fin_body(a0_ref, a1_ref, rt_ref, o_ref):
    o_ref[...] = _leaky(a0_ref[...] + a1_ref[...] + rt_ref[...])


_fin = pl.pallas_call(
    _fin_body,
    grid=(N // BM,),
    in_specs=[_row_spec, _row_spec, _row_spec],
    out_specs=pl.BlockSpec((BM, D), lambda i: (i, 0)),
    out_shape=jax.ShapeDtypeStruct((N, D), jnp.float32),
)

EB = 128


def _prep_body(src_ref, et_ref, o_ref):
    o_ref[...] = src_ref[...] * R + et_ref[...]


_prep = pl.pallas_call(
    _prep_body,
    grid=(pl.cdiv(E // EB, EB),),
    in_specs=[
        pl.BlockSpec((EB, EB), lambda i: (i, 0)),
        pl.BlockSpec((EB, EB), lambda i: (i, 0)),
    ],
    out_specs=pl.BlockSpec((EB, EB), lambda i: (i, 0)),
    out_shape=jax.ShapeDtypeStruct((E // EB, EB), jnp.int32),
)


CK = 80
EPW = E // NW
CPT = EPW // CK
G = 5
RPT = 632


def _edge_body(table_hbm, idx_hbm, dst_hbm, zeros_hbm, out_hbm,
               idxv, dstv, rows0, rows1, acc_sh, sem0, sem1):
    cid = lax.axis_index("c")
    sid = lax.axis_index("s")
    wid = sid * NC + cid
    ebase = pl.multiple_of(wid * EPW, 8)
    pltpu.sync_copy(idx_hbm.at[pl.ds(ebase, EPW)], idxv)
    pltpu.sync_copy(dst_hbm.at[wid], dstv)
    rbase = pl.multiple_of(jnp.minimum(sid * RPT, N - RPT), 8)
    pltpu.sync_copy(
        zeros_hbm.at[pl.ds(rbase, RPT)], acc_sh.at[pl.ds(rbase, RPT)]
    )

    rows = (rows0, rows1)
    sems = (sem0, sem1)

    def _gather(c, b):
        gsl = pl.ds(pl.multiple_of(c * CK, 8), CK)
        return pltpu.async_copy(table_hbm.at[idxv.at[gsl]], rows[b], sems[b])

    plsc.subcore_barrier()

    def _group(gi, carry):
        base = gi * G
        d = [_gather(base, 0), _gather(base + 1, 1)]
        for j in range(G):
            b = j % 2
            d[b].wait()
            pltpu.sync_copy(rows[b], acc_sh.at[dstv.at[base + j]], add=True)
            if j + 2 < G:
                d[b] = _gather(base + j + 2, b)
        return carry

    lax.fori_loop(0, CPT // G, _group, 0)

    plsc.subcore_barrier()
    pltpu.sync_copy(
        acc_sh.at[pl.ds(rbase, RPT)], out_hbm.at[cid, pl.ds(rbase, RPT)]
    )


_edge_agg = functools.partial(
    pl.kernel,
    out_type=jax.ShapeDtypeStruct((NC, N, D), jnp.float32),
    mesh=plsc.VectorSubcoreMesh(core_axis_name="c", subcore_axis_name="s"),
    scratch_types=[
        pltpu.VMEM((EPW,), jnp.int32),
        pltpu.VMEM((CPT, CK), jnp.int32),
        pltpu.VMEM((CK, D), jnp.float32),
        pltpu.VMEM((CK, D), jnp.float32),
        pltpu.VMEM_SHARED((N, D), jnp.float32),
        pltpu.SemaphoreType.DMA,
        pltpu.SemaphoreType.DMA,
    ],
)(_edge_body)


def kernel(node_type, edge_index, edge_type, node_emb, W_rel, W_root, b):
    nt = node_type.astype(jnp.int32).reshape(N // BM, 1, BM)
    src = edge_index[0].astype(jnp.int32).reshape(E // EB, EB)
    dst = edge_index[1].astype(jnp.int32).reshape(NW, CPT, CK)
    et = edge_type.astype(jnp.int32).reshape(E // EB, EB)
    emb = jnp.zeros((TPAD, D), jnp.float32).at[:node_emb.shape[0]].set(node_emb)
    zeros = jnp.zeros((N, D), jnp.float32)

    flat_idx = _prep(src, et).reshape(E)

    rt = None
    aggs = None
    for l in range(L):
        wr = W_rel[l].transpose(1, 0, 2).reshape(D, R * D)
        wo = W_root[l]
        bl = b[l].reshape(1, D)
        if l == 0:
            xw, rt = _mm0(nt, emb, wr, wo, bl)
        else:
            xw, rt = _mm1(aggs[0], aggs[1], rt, wr, wo, bl)
        aggs = _edge_agg(xw.reshape(N * R, D), flat_idx, dst, zeros)

    return _fin(aggs[0], aggs[1], rt)
